# baseline (speedup 1.0000x reference)
import functools

import jax
import jax.numpy as jnp
from jax import lax
from jax.experimental import pallas as pl
from jax.experimental.pallas import tpu as pltpu
from jax.experimental.pallas import tpu_sc as plsc


@functools.lru_cache(maxsize=None)
def _build_sc_call(B, H, W, NO, D):
    IMG = H * W
    NIDX = IMG + 5
    PADW = ((NIDX + 15) // 16) * 16
    NG = PADW // 16
    NW = 32
    RPW = B // NW
    R = 2
    NCH = RPW // R
    ROW = NIDX * D
    SROW = PADW * D

    off_pos0 = NO + 2
    off_pos1 = off_pos0 + H
    off_am = off_pos1 + W
    off_dir = off_am + NO
    off_prev = off_dir + 4

    mesh = plsc.VectorSubcoreMesh(core_axis_name="c", subcore_axis_name="s")

    @functools.partial(
        pl.kernel,
        mesh=mesh,
        out_type=jax.ShapeDtypeStruct((B // 8, SROW // 128, 8, 128),
                                      jnp.float32),
        compiler_params=pltpu.CompilerParams(needs_layout_passes=False,
                                             use_tc_tiling_on_sc=False),
        scratch_types=[
            pltpu.VMEM((RPW * PADW,), jnp.int32),
            pltpu.VMEM((SROW // 128, R, 128), jnp.float32),
            pltpu.VMEM((SROW // 128, R, 128), jnp.float32),
            pltpu.VMEM((89 * D,), jnp.float32),
            pltpu.VMEM((RPW * 2,), jnp.int32),
            pltpu.VMEM((RPW,), jnp.int32),
            pltpu.VMEM((RPW,), jnp.int32),
            pltpu.VMEM((RPW * NO,), jnp.float32),
            pltpu.SemaphoreType.DMA,
            pltpu.SemaphoreType.DMA,
        ],
    )
    def sc_fn(im_ref, pos_ref, dir_ref, prev_ref, task_ref, table_ref,
              out_ref, img_idx, rows0, rows1, tab_v, pos_v, dir_v,
              prev_v, task_v, so0, so1):
        wid = lax.axis_index("s") * 2 + lax.axis_index("c")
        base = wid * RPW
        rows_bufs = (rows0, rows1)
        so = (so0, so1)

        pltpu.sync_copy(im_ref.at[pl.ds(base * PADW, RPW * PADW)], img_idx)
        pltpu.sync_copy(table_ref, tab_v)
        pltpu.sync_copy(pos_ref.at[pl.ds(base * 2, RPW * 2)], pos_v)
        pltpu.sync_copy(dir_ref.at[pl.ds(base, RPW)], dir_v)
        pltpu.sync_copy(prev_ref.at[pl.ds(base, RPW)], prev_v)
        pltpu.sync_copy(task_ref.at[pl.ds(base * NO, RPW * NO)], task_v)

        iot = lax.iota(jnp.int32, 16)
        viota32 = iot * D

        for g in range(RPW // 16):
            rows = g * 16 + iot
            p0 = plsc.load_gather(pos_v, [rows * 2]) + off_pos0
            p1 = plsc.load_gather(pos_v, [rows * 2 + 1]) + off_pos1
            dd = dir_v[pl.ds(g * 16, 16)] + off_dir
            pv = prev_v[pl.ds(g * 16, 16)] + off_prev
            m = jnp.full((16,), -jnp.inf, jnp.float32)
            am = jnp.zeros((16,), jnp.int32)
            for f in range(NO):
                vals = plsc.load_gather(task_v, [rows * NO + f])
                am = jnp.where(vals > m, f, am)
                m = jnp.maximum(m, vals)
            e = rows * PADW + IMG
            plsc.store_scatter(img_idx, [e], p0)
            plsc.store_scatter(img_idx, [e + 1], p1)
            plsc.store_scatter(img_idx, [e + 2], am + off_am)
            plsc.store_scatter(img_idx, [e + 3], dd)
            plsc.store_scatter(img_idx, [e + 4], pv)

        def compute_chunk(c, b):
            flat = rows_bufs[b]
            for r in range(R):
                irow = (c * R + r) * PADW

                @plsc.parallel_loop(0, NG, unroll=2)
                def _(g):
                    offs = img_idx[pl.ds(irow + g * 16, 16)] * D
                    for jj in range(16):
                        o = offs[jj]
                        tcol = g * 4 + (jj >> 2)
                        off128 = (jj & 3) * D
                        for k in range(0, D, 16):
                            flat[tcol, r, pl.ds(off128 + k, 16)] = (
                                tab_v[pl.ds(o + k, 16)])

        def out_copies(c, b, start):
            brow = base + c * R
            tr = brow // 8
            ro = brow % 8
            src = rows_bufs[b]
            dst = out_ref.at[tr, :, pl.ds(ro, R), :]
            if start:
                pltpu.async_copy(src, dst, so[b])
            else:
                pltpu.make_async_copy(src, dst, so[b]).wait()

        def start_out(c, b):
            out_copies(c, b, True)

        def drain_out(c, b):
            out_copies(c, b, False)

        def loop_body(g, carry):
            for b in range(2):
                c = g * 2 + b

                @pl.when(g > 0)
                def _():
                    drain_out(c - 2, b)

                compute_chunk(c, b)
                start_out(c, b)
            return carry

        lax.fori_loop(0, NCH // 2, loop_body, 0)
        drain_out(NCH - 2, 0)
        drain_out(NCH - 1, 1)

    return sc_fn


def kernel(image, position, task_w, direction, prev_action, table):
    B, H, W = image.shape
    NO = task_w.shape[-1]
    D = table.shape[-1]
    IMG = H * W
    NIDX = IMG + 5
    PADW = ((NIDX + 15) // 16) * 16
    im = image.reshape(B, IMG).astype(jnp.int32)
    im_pad = jnp.pad(im, ((0, 0), (0, PADW - IMG))).reshape(-1)
    sc = _build_sc_call(B, H, W, NO, D)
    out = sc(im_pad, position.reshape(-1).astype(jnp.int32),
             direction.astype(jnp.int32), prev_action.astype(jnp.int32),
             task_w.reshape(-1).astype(jnp.float32),
             table.reshape(-1).astype(jnp.float32))
    SROW = PADW * D
    y = out.transpose(0, 2, 1, 3).reshape(B, SROW)
    return y[:, :NIDX * D]


# speedup vs baseline: 1.0144x; 1.0144x over previous
import functools

import jax
import jax.numpy as jnp
from jax import lax
from jax.experimental import pallas as pl
from jax.experimental.pallas import tpu as pltpu
from jax.experimental.pallas import tpu_sc as plsc


@functools.lru_cache(maxsize=None)
def _build_sc_call(B, H, W, NO, D, V):
    IMG = H * W
    NIDX = IMG + 5
    PADW = ((NIDX + 15) // 16) * 16
    NG = PADW // 16
    NW = 32
    RPW = B // NW
    R = 2
    NCH = RPW // R
    SROW = PADW * D
    LPT = 128 // D
    TPG = 16 // LPT

    off_pos0 = NO + 2
    off_pos1 = off_pos0 + H
    off_am = off_pos1 + W
    off_dir = off_am + NO
    off_prev = off_dir + 4

    mesh = plsc.VectorSubcoreMesh(core_axis_name="c", subcore_axis_name="s")

    @functools.partial(
        pl.kernel,
        mesh=mesh,
        out_type=jax.ShapeDtypeStruct((B // 8, SROW // 128, 8, 128),
                                      jnp.float32),
        compiler_params=pltpu.CompilerParams(needs_layout_passes=False,
                                             use_tc_tiling_on_sc=False),
        scratch_types=[
            pltpu.VMEM((RPW * PADW,), jnp.int32),
            pltpu.VMEM((SROW // 128, R, 128), jnp.float32),
            pltpu.VMEM((SROW // 128, R, 128), jnp.float32),
            pltpu.VMEM((V * D,), jnp.float32),
            pltpu.VMEM((RPW * 2,), jnp.int32),
            pltpu.VMEM((RPW,), jnp.int32),
            pltpu.VMEM((RPW,), jnp.int32),
            pltpu.VMEM((RPW * NO,), jnp.float32),
            pltpu.SemaphoreType.DMA,
            pltpu.SemaphoreType.DMA,
        ],
    )
    def sc_fn(im_ref, pos_ref, dir_ref, prev_ref, task_ref, table_ref,
              out_ref, img_idx, rows0, rows1, tab_v, pos_v, dir_v,
              prev_v, task_v, so0, so1):
        wid = lax.axis_index("s") * 2 + lax.axis_index("c")
        base = wid * RPW
        rows_bufs = (rows0, rows1)
        so = (so0, so1)

        pltpu.sync_copy(im_ref.at[pl.ds(base * PADW, RPW * PADW)], img_idx)
        pltpu.sync_copy(table_ref, tab_v)
        pltpu.sync_copy(pos_ref.at[pl.ds(base * 2, RPW * 2)], pos_v)
        pltpu.sync_copy(dir_ref.at[pl.ds(base, RPW)], dir_v)
        pltpu.sync_copy(prev_ref.at[pl.ds(base, RPW)], prev_v)
        pltpu.sync_copy(task_ref.at[pl.ds(base * NO, RPW * NO)], task_v)

        iot = lax.iota(jnp.int32, 16)

        for g in range(RPW // 16):
            rows = g * 16 + iot
            p0 = plsc.load_gather(pos_v, [rows * 2]) + off_pos0
            p1 = plsc.load_gather(pos_v, [rows * 2 + 1]) + off_pos1
            dd = dir_v[pl.ds(g * 16, 16)] + off_dir
            pv = prev_v[pl.ds(g * 16, 16)] + off_prev
            m = jnp.full((16,), -jnp.inf, jnp.float32)
            am = jnp.zeros((16,), jnp.int32)
            for f in range(NO):
                vals = plsc.load_gather(task_v, [rows * NO + f])
                am = jnp.where(vals > m, f, am)
                m = jnp.maximum(m, vals)
            e = rows * PADW + IMG
            plsc.store_scatter(img_idx, [e], p0)
            plsc.store_scatter(img_idx, [e + 1], p1)
            plsc.store_scatter(img_idx, [e + 2], am + off_am)
            plsc.store_scatter(img_idx, [e + 3], dd)
            plsc.store_scatter(img_idx, [e + 4], pv)

        def compute_chunk(c, b):
            stage = rows_bufs[b]
            for r in range(R):
                irow = (c * R + r) * PADW

                @plsc.parallel_loop(0, NG, unroll=2)
                def _(g):
                    offs = img_idx[pl.ds(irow + g * 16, 16)] * D
                    for jj in range(16):
                        o = offs[jj]
                        tcol = g * TPG + jj // LPT
                        off128 = (jj % LPT) * D
                        for k in range(0, D, 16):
                            stage[tcol, r, pl.ds(off128 + k, 16)] = (
                                tab_v[pl.ds(o + k, 16)])

        def out_copies(c, b, start):
            brow = base + c * R
            tr = brow // 8
            ro = brow % 8
            src = rows_bufs[b]
            dst = out_ref.at[tr, :, pl.ds(ro, R), :]
            if start:
                pltpu.async_copy(src, dst, so[b])
            else:
                pltpu.make_async_copy(src, dst, so[b]).wait()

        def start_out(c, b):
            out_copies(c, b, True)

        def drain_out(c, b):
            out_copies(c, b, False)

        def loop_body(g, carry):
            for b in range(2):
                c = g * 2 + b

                @pl.when(g > 0)
                def _():
                    drain_out(c - 2, b)

                compute_chunk(c, b)
                start_out(c, b)
            return carry

        lax.fori_loop(0, NCH // 2, loop_body, 0)
        drain_out(NCH - 2, 0)
        drain_out(NCH - 1, 1)

    return sc_fn


def kernel(image, position, task_w, direction, prev_action, table):
    B, H, W = image.shape
    NO = task_w.shape[-1]
    D = table.shape[-1]
    IMG = H * W
    NIDX = IMG + 5
    PADW = ((NIDX + 15) // 16) * 16
    im = image.reshape(B, IMG).astype(jnp.int32)
    im_pad = jnp.pad(im, ((0, 0), (0, PADW - IMG))).reshape(-1)
    sc = _build_sc_call(B, H, W, NO, D, table.shape[0])
    out = sc(im_pad, position.reshape(-1).astype(jnp.int32),
             direction.astype(jnp.int32), prev_action.astype(jnp.int32),
             task_w.reshape(-1).astype(jnp.float32),
             table.reshape(-1).astype(jnp.float32))
    SROW = PADW * D
    y = out.transpose(0, 2, 1, 3).reshape(B, SROW)
    return y[:, :NIDX * D]
